# Initial kernel scaffold; baseline (speedup 1.0000x reference)
import jax
import jax.numpy as jnp
from jax.experimental import pallas as pl


def kernel(x, token_table, pos_table):
    raise NotImplementedError("write your pallas kernel here")


# speedup vs baseline: 3.1070x; 3.1070x over previous
import functools

import jax
import jax.numpy as jnp
from jax import lax
from jax.experimental import pallas as pl
from jax.experimental.pallas import tpu as pltpu
from jax.experimental.pallas import tpu_sc as plsc

BATCH = 4096
SEQ = 200
EMBED = 64
NC = 2
NS = 16
NW = NC * NS
TOKENS = BATCH * SEQ
TOK_PER_W = TOKENS // NW
ROWS_PER_W = BATCH // NW


def _body(x_hbm, tok_hbm, pos_hbm, out_hbm, idx_v, rows_v, pos_v, sem):
    c = lax.axis_index("c")
    s = lax.axis_index("s")
    wid = s * NC + c
    base0 = wid * TOK_PER_W

    pltpu.sync_copy(pos_hbm, pos_v)

    def chunk(g, carry):
        base = base0 + g * SEQ
        pltpu.sync_copy(x_hbm.at[pl.ds(base, SEQ)], idx_v)
        pltpu.async_copy(tok_hbm.at[idx_v], rows_v, sem).wait()

        def row(r, carry2):
            for cc in range(EMBED // 16):
                v = pos_v[r, pl.ds(cc * 16, 16)]
                plsc.addupdate(rows_v.at[r, pl.ds(cc * 16, 16)], v)
            return carry2

        lax.fori_loop(0, SEQ, row, 0)
        pltpu.sync_copy(rows_v, out_hbm.at[pl.ds(base, SEQ)])
        return carry

    lax.fori_loop(0, ROWS_PER_W, chunk, 0)


@functools.partial(jax.jit)
def _tpemb(x_flat, token_table, pos_table):
    mesh = plsc.VectorSubcoreMesh(
        core_axis_name="c", subcore_axis_name="s", num_cores=NC, num_subcores=NS
    )
    f = pl.kernel(
        _body,
        out_type=jax.ShapeDtypeStruct((TOKENS, EMBED), jnp.float32),
        mesh=mesh,
        scratch_types=[
            pltpu.VMEM((SEQ,), jnp.int32),
            pltpu.VMEM((SEQ, EMBED), jnp.float32),
            pltpu.VMEM((SEQ, EMBED), jnp.float32),
            pltpu.SemaphoreType.DMA,
        ],
        compiler_params=pltpu.CompilerParams(use_tc_tiling_on_sc=False),
    )
    return f(x_flat, token_table, pos_table)


def kernel(x, token_table, pos_table):
    x_flat = x.reshape(-1).astype(jnp.int32)
    out = _tpemb(x_flat, token_table, pos_table)
    return out.reshape(BATCH, SEQ, EMBED)


# speedup vs baseline: 4.1134x; 1.3239x over previous
import functools

import jax
import jax.numpy as jnp
from jax import lax
from jax.experimental import pallas as pl
from jax.experimental.pallas import tpu as pltpu
from jax.experimental.pallas import tpu_sc as plsc

BATCH = 4096
SEQ = 200
EMBED = 64
NC = 2
NS = 16
NW = NC * NS
TOKENS = BATCH * SEQ
TOK_PER_W = TOKENS // NW
R = 4
C = R * SEQ
CHUNKS = TOK_PER_W // C
NLANE = 16
NVEC = EMBED // NLANE


def _body(x_hbm, tok_hbm, pos_hbm, out_hbm, idx_v, rows_v, pos_v,
          gsem0, gsem1, ssem0, ssem1):
    c = lax.axis_index("c")
    s = lax.axis_index("s")
    wid = s * NC + c
    base0 = wid * TOK_PER_W
    gsems = (gsem0, gsem1)
    ssems = (ssem0, ssem1)

    pltpu.sync_copy(pos_hbm, pos_v)

    def start_gather(g, buf):
        base = base0 + g * C
        pltpu.sync_copy(x_hbm.at[pl.ds(base, C)], idx_v.at[buf])
        return pltpu.async_copy(
            tok_hbm.at[idx_v.at[buf]], rows_v.at[buf], gsems[buf]
        )

    def start_scatter(g, buf):
        return pltpu.async_copy(
            rows_v.at[buf], out_hbm.at[pl.ds(base0 + g * C, C)], ssems[buf]
        )

    def add_pos(buf):
        @plsc.parallel_loop(0, SEQ, unroll=8)
        def _(r):
            for cc in range(NVEC):
                v = pos_v[r, pl.ds(cc * NLANE, NLANE)]
                for k in range(R):
                    plsc.addupdate(
                        rows_v.at[buf, k * SEQ + r, pl.ds(cc * NLANE, NLANE)], v
                    )

    def wait_gather(buf):
        pltpu.make_async_copy(
            tok_hbm.at[idx_v.at[buf]], rows_v.at[buf], gsems[buf]
        ).wait()

    def wait_scatter(g, buf):
        pltpu.make_async_copy(
            rows_v.at[buf], out_hbm.at[pl.ds(base0 + g * C, C)], ssems[buf]
        ).wait()

    start_gather(0, 0)
    start_gather(1, 1)
    wait_gather(0)
    add_pos(0)
    start_scatter(0, 0)

    def pair(h, carry):
        for b in range(2):
            g = 1 + h * 2 + b
            buf = 1 - b
            nxt = b
            wait_scatter(g - 1, nxt)
            start_gather(g + 1, nxt)
            wait_gather(buf)
            add_pos(buf)
            start_scatter(g, buf)
        return carry

    lax.fori_loop(0, (CHUNKS - 2) // 2, pair, 0)

    g_last = CHUNKS - 1
    wait_scatter(g_last - 1, 0)
    wait_gather(1)
    add_pos(1)
    start_scatter(g_last, 1)
    wait_scatter(g_last, 1)


@functools.partial(jax.jit)
def _tpemb(x_flat, token_table, pos_table):
    mesh = plsc.VectorSubcoreMesh(
        core_axis_name="c", subcore_axis_name="s", num_cores=NC, num_subcores=NS
    )
    f = pl.kernel(
        _body,
        out_type=jax.ShapeDtypeStruct((TOKENS, EMBED), jnp.float32),
        mesh=mesh,
        scratch_types=[
            pltpu.VMEM((2, C), jnp.int32),
            pltpu.VMEM((2, C, EMBED), jnp.float32),
            pltpu.VMEM((SEQ, EMBED), jnp.float32),
            pltpu.SemaphoreType.DMA,
            pltpu.SemaphoreType.DMA,
            pltpu.SemaphoreType.DMA,
            pltpu.SemaphoreType.DMA,
        ],
        compiler_params=pltpu.CompilerParams(use_tc_tiling_on_sc=False),
    )
    return f(x_flat, token_table, pos_table)


def kernel(x, token_table, pos_table):
    x_flat = x.reshape(-1).astype(jnp.int32)
    out = _tpemb(x_flat, token_table, pos_table)
    return out.reshape(BATCH, SEQ, EMBED)
